# baseline (device time: 52680 ns/iter reference)
import contextlib
import os

import jax
import jax.numpy as jnp
from jax import lax
from jax.experimental import pallas as pl
from jax.experimental.pallas import tpu as pltpu

N_DEV = 4
NSUB = 3
_PROFILE = os.environ.get("PROFILE_SCOPES") == "1"


def _scope(name):
    return jax.named_scope(name) if _PROFILE else contextlib.nullcontext()


def kernel(A, B):
    m, k = A.shape
    _, n = B.shape
    mh = m // 2
    mq = m // 4
    nh = n // 2
    nq = nh // NSUB

    ORDER = [(b, c) for c in range(NSUB) for b in range(2)]
    NMSG = len(ORDER)

    def body(
        a_hbm, b_hbm, out_ref, a_ref, b_vmem, bbf_ref, res_ref,
        s1_send, s1_recv, acc_ref, s2_send, s2_recv,
        s1_sems, r1_sems, s2_sems, r2_sems,
        s3a_sems, r3a_sems, s3b_sems, r3b_sems, s4_sems, r4_sems,
        out_sems, in_sems,
    ):
        p = lax.axis_index("i")
        nbr_a = jnp.bitwise_xor(p, 1)
        nbr_b = 3 - p

        a_cpy = pltpu.make_async_copy(a_hbm, a_ref, in_sems.at[0])
        a_cpy.start()
        b_cpy = pltpu.make_async_copy(b_hbm, b_vmem, in_sems.at[1])
        b_cpy.start()

        with _scope("barrier"):
            barrier_sem = pltpu.get_barrier_semaphore()
            for nbr in (nbr_a, nbr_b):
                pl.semaphore_signal(
                    barrier_sem, inc=1,
                    device_id=(nbr,), device_id_type=pl.DeviceIdType.MESH,
                )
            pl.semaphore_wait(barrier_sem, 2)

        with _scope("wait_inputs"):
            a_cpy.wait()
            b_cpy.wait()

        def params(b):
            if b == 0:
                p1, p2 = nbr_a, nbr_b
                half_lo = jnp.logical_or(p == 0, p == 3)
                q_lo = p < 2
            else:
                p1, p2 = nbr_b, nbr_a
                half_lo = p < 2
                q_lo = lax.rem(p, 2) == 0
            half_start = jnp.where(half_lo, 0, mh)
            rel_q = jnp.where(q_lo, 0, mq)
            return p1, p2, half_start, rel_q

        P = [params(0), params(1)]

        def col0(b, c):
            return b * nh + c * nq

        def mm(row_start, b, c):
            a = a_ref[pl.ds(row_start, mh), :].astype(jnp.bfloat16)
            return jnp.dot(
                a, bbf_ref[:, col0(b, c):col0(b, c) + nq],
                preferred_element_type=jnp.float32,
            )

        def copy(src, dst, send_sems, recv_sems, i, dev):
            return pltpu.make_async_remote_copy(
                src_ref=src, dst_ref=dst,
                send_sem=send_sems.at[i], recv_sem=recv_sems.at[i],
                device_id=(dev,), device_id_type=pl.DeviceIdType.MESH,
            )

        rs1 = {}
        for i, (b, c) in enumerate(ORDER):
            p1, p2, half_start, rel_q = P[b]
            with _scope(f"mm_send#i={i}"):
                bbf_ref[:, col0(b, c):col0(b, c) + nq] = b_vmem[
                    :, col0(b, c):col0(b, c) + nq
                ].astype(jnp.bfloat16)
                s1_send[b, c] = mm(mh - half_start, b, c).astype(jnp.bfloat16)
            r = copy(s1_send.at[b, c], s1_recv.at[b, c], s1_sems, r1_sems, i, p1)
            r.start()
            rs1[b, c] = r

        for i, (b, c) in enumerate(ORDER):
            _, _, half_start, _ = P[b]
            with _scope(f"mm_keep#i={i}"):
                acc_ref[b, c] = mm(half_start, b, c)

        rs2 = {}
        for i, (b, c) in enumerate(ORDER):
            p1, p2, half_start, rel_q = P[b]
            with _scope(f"wait_rs1#i={i}"):
                rs1[b, c].wait_recv()
            with _scope(f"add1#i={i}"):
                sq = pl.ds(mq - rel_q, mq)
                s2_send[b, c] = (
                    acc_ref[b, c, sq, :]
                    + s1_recv[b, c, sq, :].astype(jnp.float32)
                ).astype(jnp.bfloat16)
            r = copy(s2_send.at[b, c], s2_recv.at[b, c], s2_sems, r2_sems, i, p2)
            r.start()
            rs2[b, c] = r

        def to_hbm(row_start, rows, b, c, j):
            cpy = pltpu.make_async_copy(
                res_ref.at[pl.ds(row_start, rows), col0(b, c):col0(b, c) + nq],
                out_ref.at[pl.ds(row_start, rows), col0(b, c):col0(b, c) + nq],
                out_sems.at[3 * (c * 2 + b) + j],
            )
            cpy.start()
            return cpy

        hbm = []
        ag3a = {}
        ag3b = {}
        for i, (b, c) in enumerate(ORDER):
            p1, p2, half_start, rel_q = P[b]
            with _scope(f"wait_rs2#i={i}"):
                rs2[b, c].wait_recv()
            with _scope(f"relu#i={i}"):
                kq = pl.ds(rel_q, mq)
                q_sum = (
                    acc_ref[b, c, kq, :]
                    + s1_recv[b, c, kq, :].astype(jnp.float32)
                    + s2_recv[b, c].astype(jnp.float32)
                )
                q_start = half_start + rel_q
                res_ref[
                    pl.ds(q_start, mq), col0(b, c):col0(b, c) + nq
                ] = jnp.maximum(q_sum, 0.0).astype(jnp.bfloat16)
            q_slice = res_ref.at[pl.ds(q_start, mq), col0(b, c):col0(b, c) + nq]
            r = copy(q_slice, q_slice, s3a_sems, r3a_sems, i, p2)
            r.start()
            ag3a[b, c] = r
            r = copy(q_slice, q_slice, s3b_sems, r3b_sems, i, p1)
            r.start()
            ag3b[b, c] = r
            hbm.append(to_hbm(q_start, mq, b, c, 0))

        ag4 = {}
        for i, (b, c) in enumerate(ORDER):
            p1, p2, half_start, rel_q = P[b]
            with _scope(f"wait_ag3a#i={i}"):
                ag3a[b, c].wait_recv()
            f_start = half_start + mq - rel_q
            f_slice = res_ref.at[pl.ds(f_start, mq), col0(b, c):col0(b, c) + nq]
            r = copy(f_slice, f_slice, s4_sems, r4_sems, i, p1)
            r.start()
            ag4[b, c] = r
            hbm.append(to_hbm(f_start, mq, b, c, 1))

        for i, (b, c) in enumerate(ORDER):
            p1, p2, half_start, rel_q = P[b]
            with _scope(f"wait_tail#i={i}"):
                ag3b[b, c].wait_recv()
                ag4[b, c].wait_recv()
            hbm.append(to_hbm(mh - half_start, mh, b, c, 2))

        with _scope("drain"):
            for group in (rs1, rs2, ag3a, ag3b, ag4):
                for r in group.values():
                    r.wait_send()
            for cpy in hbm:
                cpy.wait()

    return pl.pallas_call(
        body,
        out_shape=jax.ShapeDtypeStruct((m, n), jnp.bfloat16),
        in_specs=[
            pl.BlockSpec(memory_space=pl.ANY),
            pl.BlockSpec(memory_space=pl.ANY),
        ],
        out_specs=pl.BlockSpec(memory_space=pl.ANY),
        scratch_shapes=[
            pltpu.VMEM((m, k), jnp.float32),
            pltpu.VMEM((k, n), jnp.float32),
            pltpu.VMEM((k, n), jnp.bfloat16),
            pltpu.VMEM((m, n), jnp.bfloat16),
            pltpu.VMEM((2, NSUB, mh, nq), jnp.bfloat16),
            pltpu.VMEM((2, NSUB, mh, nq), jnp.bfloat16),
            pltpu.VMEM((2, NSUB, mh, nq), jnp.float32),
            pltpu.VMEM((2, NSUB, mq, nq), jnp.bfloat16),
            pltpu.VMEM((2, NSUB, mq, nq), jnp.bfloat16),
        ] + [pltpu.SemaphoreType.DMA((NMSG,))] * 10
        + [
            pltpu.SemaphoreType.DMA((3 * NMSG,)),
            pltpu.SemaphoreType.DMA((2,)),
        ],
        compiler_params=pltpu.CompilerParams(collective_id=0),
    )(A, B)


# device time: 50644 ns/iter; 1.0402x vs baseline; 1.0402x over previous
import contextlib
import os

import jax
import jax.numpy as jnp
from jax import lax
from jax.experimental import pallas as pl
from jax.experimental.pallas import tpu as pltpu

N_DEV = 4
NSUB = 3
_PROFILE = os.environ.get("PROFILE_SCOPES") == "1"


def _scope(name):
    return jax.named_scope(name) if _PROFILE else contextlib.nullcontext()


def kernel(A, B):
    m, k = A.shape
    _, n = B.shape
    mh = m // 2
    mq = m // 4
    nh = n // 2
    nq = nh // NSUB

    ORDER = [(b, c) for c in range(NSUB) for b in range(2)]
    NMSG = len(ORDER)

    def body(
        a_ref, b_ref, out_ref,
        bbf_ref, s1_send, s1_recv, acc_ref, s2_send, s2_recv,
        s1_sems, r1_sems, s2_sems, r2_sems,
        s3a_sems, r3a_sems, s3b_sems, r3b_sems, s4_sems, r4_sems,
    ):
        p = lax.axis_index("i")
        nbr_a = jnp.bitwise_xor(p, 1)
        nbr_b = 3 - p

        with _scope("barrier"):
            barrier_sem = pltpu.get_barrier_semaphore()
            for nbr in (nbr_a, nbr_b):
                pl.semaphore_signal(
                    barrier_sem, inc=1,
                    device_id=(nbr,), device_id_type=pl.DeviceIdType.MESH,
                )
            pl.semaphore_wait(barrier_sem, 2)

        def params(b):
            if b == 0:
                p1, p2 = nbr_a, nbr_b
                half_lo = jnp.logical_or(p == 0, p == 3)
                q_lo = p < 2
            else:
                p1, p2 = nbr_b, nbr_a
                half_lo = p < 2
                q_lo = lax.rem(p, 2) == 0
            half_start = jnp.where(half_lo, 0, mh)
            rel_q = jnp.where(q_lo, 0, mq)
            return p1, p2, half_start, rel_q

        P = [params(0), params(1)]

        def col0(b, c):
            return b * nh + c * nq

        def mm(row_start, b, c):
            a = a_ref[pl.ds(row_start, mh), :].astype(jnp.bfloat16)
            return jnp.dot(
                a, bbf_ref[:, col0(b, c):col0(b, c) + nq],
                preferred_element_type=jnp.float32,
            )

        def copy(src, dst, send_sems, recv_sems, i, dev):
            return pltpu.make_async_remote_copy(
                src_ref=src, dst_ref=dst,
                send_sem=send_sems.at[i], recv_sem=recv_sems.at[i],
                device_id=(dev,), device_id_type=pl.DeviceIdType.MESH,
            )

        rs1 = {}
        for i, (b, c) in enumerate(ORDER):
            p1, p2, half_start, rel_q = P[b]
            with _scope(f"mm_send#i={i}"):
                bbf_ref[:, col0(b, c):col0(b, c) + nq] = b_ref[
                    :, col0(b, c):col0(b, c) + nq
                ].astype(jnp.bfloat16)
                s1_send[b, c] = mm(mh - half_start, b, c).astype(jnp.bfloat16)
            r = copy(s1_send.at[b, c], s1_recv.at[b, c], s1_sems, r1_sems, i, p1)
            r.start()
            rs1[b, c] = r

        for i, (b, c) in enumerate(ORDER):
            _, _, half_start, _ = P[b]
            with _scope(f"mm_keep#i={i}"):
                acc_ref[b, c] = mm(half_start, b, c)

        rs2 = {}
        for i, (b, c) in enumerate(ORDER):
            p1, p2, half_start, rel_q = P[b]
            with _scope(f"wait_rs1#i={i}"):
                rs1[b, c].wait_recv()
            with _scope(f"add1#i={i}"):
                sq = pl.ds(mq - rel_q, mq)
                s2_send[b, c] = (
                    acc_ref[b, c, sq, :]
                    + s1_recv[b, c, sq, :].astype(jnp.float32)
                ).astype(jnp.bfloat16)
            r = copy(s2_send.at[b, c], s2_recv.at[b, c], s2_sems, r2_sems, i, p2)
            r.start()
            rs2[b, c] = r

        ag3a = {}
        ag3b = {}
        for i, (b, c) in enumerate(ORDER):
            p1, p2, half_start, rel_q = P[b]
            with _scope(f"wait_rs2#i={i}"):
                rs2[b, c].wait_recv()
            with _scope(f"relu#i={i}"):
                kq = pl.ds(rel_q, mq)
                q_sum = (
                    acc_ref[b, c, kq, :]
                    + s1_recv[b, c, kq, :].astype(jnp.float32)
                    + s2_recv[b, c].astype(jnp.float32)
                )
                q_start = half_start + rel_q
                out_ref[
                    pl.ds(q_start, mq), col0(b, c):col0(b, c) + nq
                ] = jnp.maximum(q_sum, 0.0).astype(jnp.bfloat16)
            q_slice = out_ref.at[pl.ds(q_start, mq), col0(b, c):col0(b, c) + nq]
            r = copy(q_slice, q_slice, s3a_sems, r3a_sems, i, p2)
            r.start()
            ag3a[b, c] = r
            r = copy(q_slice, q_slice, s3b_sems, r3b_sems, i, p1)
            r.start()
            ag3b[b, c] = r

        ag4 = {}
        for i, (b, c) in enumerate(ORDER):
            p1, p2, half_start, rel_q = P[b]
            with _scope(f"wait_ag3a#i={i}"):
                ag3a[b, c].wait_recv()
            f_start = half_start + mq - rel_q
            f_slice = out_ref.at[pl.ds(f_start, mq), col0(b, c):col0(b, c) + nq]
            r = copy(f_slice, f_slice, s4_sems, r4_sems, i, p1)
            r.start()
            ag4[b, c] = r

        for i, (b, c) in enumerate(ORDER):
            with _scope(f"wait_tail#i={i}"):
                ag3b[b, c].wait_recv()
                ag4[b, c].wait_recv()

        with _scope("drain"):
            for group in (rs1, rs2, ag3a, ag3b, ag4):
                for r in group.values():
                    r.wait_send()

    return pl.pallas_call(
        body,
        out_shape=jax.ShapeDtypeStruct((m, n), jnp.bfloat16),
        in_specs=[
            pl.BlockSpec(memory_space=pltpu.VMEM),
            pl.BlockSpec(memory_space=pltpu.VMEM),
        ],
        out_specs=pl.BlockSpec(memory_space=pltpu.VMEM),
        scratch_shapes=[
            pltpu.VMEM((k, n), jnp.bfloat16),
            pltpu.VMEM((2, NSUB, mh, nq), jnp.bfloat16),
            pltpu.VMEM((2, NSUB, mh, nq), jnp.bfloat16),
            pltpu.VMEM((2, NSUB, mh, nq), jnp.float32),
            pltpu.VMEM((2, NSUB, mq, nq), jnp.bfloat16),
            pltpu.VMEM((2, NSUB, mq, nq), jnp.bfloat16),
        ] + [pltpu.SemaphoreType.DMA((NMSG,))] * 10,
        compiler_params=pltpu.CompilerParams(collective_id=0),
    )(A, B)
